# baseline (speedup 1.0000x reference)
import jax
import jax.numpy as jnp
from jax import lax
from jax.experimental import pallas as pl
from jax.experimental.pallas import tpu as pltpu
from jax.experimental.pallas import tpu_sc as plsc

BATCH = 16384
EMBED_DIM = 32
NUM_CORES = 2
NUM_SUBCORES = 16
LANES = 16
NUM_WORKERS = NUM_CORES * NUM_SUBCORES
BPW = BATCH // NUM_WORKERS


def _body(uid_hbm, iid_hbm, uemb_hbm, iemb_hbm, ubias_hbm, ibias_hbm,
          out_hbm, uidx_v, iidx_v, ue_v, ie_v, ub_v, ib_v, out_v, sem):
    wid = lax.axis_index("s") * NUM_CORES + lax.axis_index("c")
    base = wid * BPW

    pltpu.sync_copy(uid_hbm.at[pl.ds(base, BPW)], uidx_v)
    pltpu.sync_copy(iid_hbm.at[pl.ds(base, BPW)], iidx_v)

    cp_ue = pltpu.async_copy(uemb_hbm.at[uidx_v], ue_v, sem)
    cp_ie = pltpu.async_copy(iemb_hbm.at[iidx_v], ie_v, sem)
    cp_ub = pltpu.async_copy(ubias_hbm.at[uidx_v], ub_v, sem)
    cp_ib = pltpu.async_copy(ibias_hbm.at[iidx_v], ib_v, sem)
    cp_ue.wait()
    cp_ie.wait()
    cp_ub.wait()
    cp_ib.wait()

    def group_body(g, carry):
        row0 = g * LANES
        rows = row0 + lax.iota(jnp.int32, LANES)
        acc = ub_v[pl.ds(row0, LANES)] + ib_v[pl.ds(row0, LANES)]
        for d in range(EMBED_DIM):
            cols = jnp.full((LANES,), d, jnp.int32)
            u = plsc.load_gather(ue_v, [rows, cols])
            i = plsc.load_gather(ie_v, [rows, cols])
            acc = acc + u * i
        out_v[pl.ds(row0, LANES)] = acc
        return carry

    lax.fori_loop(0, BPW // LANES, group_body, 0)

    pltpu.sync_copy(out_v, out_hbm.at[pl.ds(base, BPW)])


@jax.jit
def _mf_scores(uid, iid, uemb, iemb, ubias, ibias):
    mesh = plsc.VectorSubcoreMesh(core_axis_name="c", subcore_axis_name="s")
    return pl.kernel(
        _body,
        out_type=jax.ShapeDtypeStruct((BATCH,), jnp.float32),
        mesh=mesh,
        compiler_params=pltpu.CompilerParams(
            needs_layout_passes=False, use_tc_tiling_on_sc=False),
        scratch_types=[
            pltpu.VMEM((BPW,), jnp.int32),
            pltpu.VMEM((BPW,), jnp.int32),
            pltpu.VMEM((BPW, EMBED_DIM), jnp.float32),
            pltpu.VMEM((BPW, EMBED_DIM), jnp.float32),
            pltpu.VMEM((BPW,), jnp.float32),
            pltpu.VMEM((BPW,), jnp.float32),
            pltpu.VMEM((BPW,), jnp.float32),
            pltpu.SemaphoreType.DMA,
        ],
    )(uid, iid, uemb, iemb, ubias, ibias)


def kernel(user_ids, item_ids, user_emb, item_emb, user_bias, item_bias):
    uid = user_ids.astype(jnp.int32)
    iid = item_ids.astype(jnp.int32)
    ub = user_bias.reshape(-1)
    ib = item_bias.reshape(-1)
    return _mf_scores(uid, iid, user_emb, item_emb, ub, ib)
